# baseline (speedup 1.0000x reference)
import numpy as np

import jax
import jax.numpy as jnp
from jax import lax
from jax.experimental import pallas as pl
from jax.experimental.pallas import tpu as pltpu
from jax.experimental.pallas import tpu_sc as plsc

_D = 128
_DW = _D // 2
_CB = 128
_R0 = 256
_R2 = 16
_NC = 2
_NS = 16
_NW = _NC * _NS

_MASK = -65536


def _compute_idx(xv, i3, kb):
    i0, i1, i2 = i3
    iota = lax.iota(jnp.int32, 16)
    rep = kb + iota
    for g in range(_CB // 16):
        sl = pl.ds(g * 16, 16)
        rg = rep + g * 16
        i0[sl] = ((xv[0, sl] * 2 + xv[9, sl]) * 2 + xv[10, sl]
                  + (rg & (_R0 - 1)) * 176)
        i1[sl] = ((xv[1, sl] * 11 + xv[2, sl]) * 11
                  + xv[3, sl]) * 11 + xv[4, sl]
        i2[sl] = (((xv[5, sl] * 6 + xv[6, sl]) * 6
                   + xv[7, sl]) * 5 + xv[8, sl]
                  + (rg & (_R2 - 1)) * 1980)


def _f32(w):
    return plsc.bitcast(w, jnp.float32)


def _accum(ga, gb, gc, ob):
    def row(r, _):
        for j in range(_DW // 16):
            sl = pl.ds(j * 16, 16)
            a = ga[r, sl]
            b = gb[r, sl]
            c = gc[r, sl]
            lo = (_f32(lax.shift_left(a, 16)) + _f32(lax.shift_left(b, 16))
                  + _f32(lax.shift_left(c, 16)))
            hi = _f32(a & _MASK) + _f32(b & _MASK) + _f32(c & _MASK)
            ob[r, pl.ds(j * 32, 16)] = lo
            ob[r, pl.ds(j * 32 + 16, 16)] = hi
        return ()

    lax.fori_loop(0, _CB, row, (), unroll=4)


def _body(xT, t0, t1, t2, out,
          xv0, xv1, i00, i01, i02, i10, i11, i12,
          ga0, gb0, gc0, ga1, gb1, gc1, o0, o1,
          xs0, xs1, gs0, gs1, os0, os1):
    E = xT.shape[1]
    nchunks = E // _CB
    wid = lax.axis_index("s") * _NC + lax.axis_index("c")
    nk = (nchunks - wid + _NW - 1) // _NW
    xvs = (xv0, xv1)
    idx = ((i00, i01, i02), (i10, i11, i12))
    gbufs = ((ga0, gb0, gc0), (ga1, gb1, gc1))
    obufs = (o0, o1)
    xsem = (xs0, xs1)
    gsem = (gs0, gs1)
    osem = (os0, os1)
    tables = (t0, t1, t2)

    def ebase(k):
        return (wid + k * _NW) * _CB

    def x_copy(k, b):
        return pltpu.make_async_copy(
            xT.at[:, pl.ds(ebase(k), _CB)], xvs[b], xsem[b])

    def gathers(b, do_start):
        cps = [pltpu.make_async_copy(tables[t].at[idx[b][t]], gbufs[b][t],
                                     gsem[b]) for t in range(3)]
        for cp in cps:
            if do_start:
                cp.start()
            else:
                cp.wait()

    def out_copy(k, b, do_start):
        cp = pltpu.make_async_copy(
            obufs[b], out.at[pl.ds(ebase(k), _CB), :], osem[b])
        if do_start:
            cp.start()
        else:
            cp.wait()

    x_copy(0, 0).start()
    x_copy(0, 0).wait()
    _compute_idx(xvs[0], idx[0], ebase(0))
    gathers(0, True)
    x_copy(1, 1).start()

    def step(k, b):
        bn = 1 - b

        @pl.when(k >= 1)
        def _():
            out_copy(k - 1, bn, False)

        @pl.when(k + 1 < nk)
        def _():
            x_copy(k + 1, bn).wait()
            _compute_idx(xvs[bn], idx[bn], ebase(k + 1))
            gathers(bn, True)

        @pl.when(k + 2 < nk)
        def _():
            x_copy(k + 2, b).start()

        gathers(b, False)
        _accum(gbufs[b][0], gbufs[b][1], gbufs[b][2], obufs[b])
        out_copy(k, b, True)

    def pair(p, _):
        k0 = 2 * p

        @pl.when(k0 < nk)
        def _():
            step(k0, 0)

        @pl.when(k0 + 1 < nk)
        def _():
            step(k0 + 1, 1)

        return ()

    lax.fori_loop(0, (nk + 1) // 2, pair, ())

    @pl.when((nk - 1) % 2 == 0)
    def _():
        out_copy(nk - 1, 0, False)

    @pl.when((nk - 1) % 2 == 1)
    def _():
        out_copy(nk - 1, 1, False)


def _pack_perm():
    perm = np.empty(_D, np.int64)
    for j in range(_D // 32):
        for i in range(16):
            perm[32 * j + 2 * i] = 32 * j + i
            perm[32 * j + 2 * i + 1] = 32 * j + 16 + i
    return perm


def _pack(t):
    tb = t[:, _pack_perm()].astype(jnp.bfloat16)
    return lax.bitcast_convert_type(tb.reshape(t.shape[0], _DW, 2),
                                    jnp.int32)


def kernel(x, W0, W1, W2, W3, W4, W5, W6, W7, W8, W9, W10):
    E = x.shape[0]
    t0 = (W0[:, None, None, :] + W9[None, :, None, :]
          + W10[None, None, :, :]).reshape(-1, _D)
    t1 = (W1[:, None, None, None, :] + W2[None, :, None, None, :]
          + W3[None, None, :, None, :]
          + W4[None, None, None, :, :]).reshape(-1, _D)
    t2 = (W5[:, None, None, None, :] + W6[None, :, None, None, :]
          + W7[None, None, :, None, :]
          + W8[None, None, None, :, :]).reshape(-1, _D)
    t0 = _pack(jnp.tile(t0, (_R0, 1)))
    t1 = _pack(t1)
    t2 = _pack(jnp.tile(t2, (_R2, 1)))
    xT = x.T

    mesh = plsc.VectorSubcoreMesh(core_axis_name="c", subcore_axis_name="s")
    run = pl.kernel(
        _body,
        out_type=jax.ShapeDtypeStruct((E, _D), jnp.float32),
        mesh=mesh,
        compiler_params=pltpu.CompilerParams(needs_layout_passes=False,
                                             use_tc_tiling_on_sc=False),
        scratch_types=[
            pltpu.VMEM((11, _CB), jnp.int32),
            pltpu.VMEM((11, _CB), jnp.int32),
            pltpu.VMEM((_CB,), jnp.int32),
            pltpu.VMEM((_CB,), jnp.int32),
            pltpu.VMEM((_CB,), jnp.int32),
            pltpu.VMEM((_CB,), jnp.int32),
            pltpu.VMEM((_CB,), jnp.int32),
            pltpu.VMEM((_CB,), jnp.int32),
            pltpu.VMEM((_CB, _DW), jnp.int32),
            pltpu.VMEM((_CB, _DW), jnp.int32),
            pltpu.VMEM((_CB, _DW), jnp.int32),
            pltpu.VMEM((_CB, _DW), jnp.int32),
            pltpu.VMEM((_CB, _DW), jnp.int32),
            pltpu.VMEM((_CB, _DW), jnp.int32),
            pltpu.VMEM((_CB, _D), jnp.float32),
            pltpu.VMEM((_CB, _D), jnp.float32),
            pltpu.SemaphoreType.DMA,
            pltpu.SemaphoreType.DMA,
            pltpu.SemaphoreType.DMA,
            pltpu.SemaphoreType.DMA,
            pltpu.SemaphoreType.DMA,
            pltpu.SemaphoreType.DMA,
        ],
    )
    return run(xT, t0, t1, t2)
